# baseline (speedup 1.0000x reference)
import functools

import jax
import jax.numpy as jnp
from jax import lax
from jax.experimental import pallas as pl
from jax.experimental.pallas import tpu as pltpu
from jax.experimental.pallas import tpu_sc as plsc

K_PER_ROW = 64
NC = 2
NS = 16
NW = NC * NS


def _encode_kernel(x_ref, w_ref, be_ref, bd_ref, out_ref):
    xb = x_ref[...] - bd_ref[...]
    acc = lax.dot_general(
        xb, w_ref[...], (((1,), (1,)), ((), ())),
        preferred_element_type=jnp.float32,
    )
    out_ref[...] = jnp.maximum(acc + be_ref[...], 0.0)


def _sc_hist_body(match_shift, bin_shift, nbins, rows_per_w, row_words,
                  acts, state, hists,
                  buf0, buf1, lo_v, hist_v, merged_v, sem0, sem1):
    c = lax.axis_index("c")
    s = lax.axis_index("s")
    wid = s * NC + c
    base_row = wid * rows_per_w

    pltpu.sync_copy(state.at[0, pl.ds(0, 16)], lo_v)
    lo_vec = lo_v[...]

    zeros16 = jnp.zeros((16,), jnp.int32)
    ones16 = jnp.ones((16,), jnp.int32)
    iota16 = lax.iota(jnp.int32, 16)

    def zbody(i, _):
        hist_v[pl.ds(i * 16, 16)] = zeros16
        return 0

    lax.fori_loop(0, nbins, zbody, 0)

    def start(buf, sem, step):
        st = jnp.minimum(step, rows_per_w - 1)
        pltpu.make_async_copy(acts.at[base_row + st], buf, sem).start()

    def wait(buf, sem, step):
        st = jnp.minimum(step, rows_per_w - 1)
        pltpu.make_async_copy(acts.at[base_row + st], buf, sem).wait()

    def process(buf):
        def ibody(i, _):
            off = i * 64
            for u in range(4):
                v = buf[pl.ds(off + u * 16, 16)]
                idx = ((v >> bin_shift) & (nbins - 1)) * 16 + iota16
                if match_shift is None:
                    plsc.addupdate_scatter(hist_v, [idx], ones16)
                else:
                    m = ((v ^ lo_vec) >> match_shift) == 0
                    plsc.addupdate_scatter(hist_v, [idx], ones16, mask=m)
            return 0

        lax.fori_loop(0, row_words // 64, ibody, 0)

    start(buf0, sem0, 0)
    start(buf1, sem1, 1)

    def obody(g, _):
        step0 = g * 2
        wait(buf0, sem0, step0)
        process(buf0)
        start(buf0, sem0, step0 + 2)
        wait(buf1, sem1, step0 + 1)
        process(buf1)
        start(buf1, sem1, step0 + 3)
        return 0

    lax.fori_loop(0, rows_per_w // 2, obody, 0)
    wait(buf0, sem0, rows_per_w - 1)
    wait(buf1, sem1, rows_per_w - 1)

    def mbody(g, _):
        bidx = g * 256 + iota16 * 16
        acc = plsc.load_gather(hist_v, [bidx])
        for l in range(1, 16):
            acc = acc + plsc.load_gather(hist_v, [bidx + l])
        merged_v[pl.ds(g * 16, 16)] = acc
        return 0

    lax.fori_loop(0, nbins // 16, mbody, 0)
    pltpu.sync_copy(merged_v, hists.at[wid])


def _sc_hist_pass(acts_i32, state_vec, match_shift, bin_shift, nbins):
    B, D = acts_i32.shape
    rows_per_w = B // NW
    mesh = plsc.VectorSubcoreMesh(core_axis_name="c", subcore_axis_name="s")
    fn = functools.partial(
        pl.kernel,
        out_type=jax.ShapeDtypeStruct((NW, nbins), jnp.int32),
        mesh=mesh,
        compiler_params=pltpu.CompilerParams(needs_layout_passes=False),
        scratch_types=[
            pltpu.VMEM((D,), jnp.int32),
            pltpu.VMEM((D,), jnp.int32),
            pltpu.VMEM((16,), jnp.int32),
            pltpu.VMEM((nbins * 16,), jnp.int32),
            pltpu.VMEM((nbins,), jnp.int32),
            pltpu.SemaphoreType.DMA,
            pltpu.SemaphoreType.DMA,
        ],
    )(functools.partial(_sc_hist_body, match_shift, bin_shift, nbins,
                        rows_per_w, D))
    return fn(acts_i32, state_vec)


def _scan_kernel(width, nbins, hists_ref, state_ref, new_smem_ref, new_vec_ref):
    lo = state_ref[0, 0]
    K = state_ref[0, 1]
    h = jnp.sum(hists_ref[...], axis=0)
    R = nbins // 128
    h2f = h.reshape(R, 128).astype(jnp.float32)
    cmaskf = (
        lax.broadcasted_iota(jnp.int32, (128, 128), 0)
        >= lax.broadcasted_iota(jnp.int32, (128, 128), 1)
    ).astype(jnp.float32)
    ws = lax.dot_general(
        h2f, cmaskf, (((1,), (0,)), ((), ())),
        precision=lax.Precision.HIGHEST,
        preferred_element_type=jnp.float32,
    )
    rowtot = jnp.sum(h2f, axis=1, keepdims=True)
    rmaskf = (
        lax.broadcasted_iota(jnp.int32, (R, R), 1)
        > lax.broadcasted_iota(jnp.int32, (R, R), 0)
    ).astype(jnp.float32)
    rs = lax.dot_general(
        rmaskf, rowtot, (((1,), (0,)), ((), ())),
        precision=lax.Precision.HIGHEST,
        preferred_element_type=jnp.float32,
    )
    S = ws + rs
    Kf = K.astype(jnp.float32)
    b = jnp.sum((S >= Kf).astype(jnp.int32)) - 1
    s_next = jnp.maximum(
        jnp.max(jnp.where(S < Kf, S, -1.0)).astype(jnp.int32), 0
    )
    new_lo = lo + b * width
    new_k = K - s_next
    new_smem_ref[0, 0] = new_lo
    new_smem_ref[0, 1] = new_k
    new_vec_ref[...] = jnp.full((8, 128), new_lo, jnp.int32)


def _scan(hists, state_smem, width, nbins):
    return pl.pallas_call(
        functools.partial(_scan_kernel, width, nbins),
        in_specs=[
            pl.BlockSpec((NW, nbins), lambda: (0, 0)),
            pl.BlockSpec(memory_space=pltpu.SMEM),
        ],
        out_specs=[
            pl.BlockSpec(memory_space=pltpu.SMEM),
            pl.BlockSpec((8, 128), lambda: (0, 0)),
        ],
        out_shape=[
            jax.ShapeDtypeStruct((1, 8), jnp.int32),
            jax.ShapeDtypeStruct((8, 128), jnp.int32),
        ],
    )(hists, state_smem)


def _decode_kernel(thr_ref, acts_ref, w_ref, bd_ref, out_ref):
    k = pl.program_id(1)
    thr = thr_ref[0, 0]
    a = acts_ref[...]
    bits = lax.bitcast_convert_type(a, jnp.int32)
    enc = jnp.where(bits >= thr, a, 0.0)
    part = lax.dot_general(
        enc, w_ref[...], (((1,), (1,)), ((), ())),
        preferred_element_type=jnp.float32,
    )

    @pl.when(k == 0)
    def _first():
        out_ref[...] = part + bd_ref[...]

    @pl.when(k != 0)
    def _acc():
        out_ref[...] += part


def kernel(x, W_enc, b_enc, W_dec, b_dec):
    B, A = x.shape
    D = W_enc.shape[0]
    K_total = K_PER_ROW * B

    BT = min(512, B)
    DT = min(2048, D)
    acts = pl.pallas_call(
        _encode_kernel,
        grid=(D // DT, B // BT),
        in_specs=[
            pl.BlockSpec((BT, A), lambda j, i: (i, 0)),
            pl.BlockSpec((DT, A), lambda j, i: (j, 0)),
            pl.BlockSpec((1, DT), lambda j, i: (0, j)),
            pl.BlockSpec((1, A), lambda j, i: (0, 0)),
        ],
        out_specs=pl.BlockSpec((BT, DT), lambda j, i: (i, j)),
        out_shape=jax.ShapeDtypeStruct((B, D), jnp.float32),
    )(x, W_enc, b_enc.reshape(1, D), b_dec.reshape(1, A))

    acts_i32 = lax.bitcast_convert_type(acts, jnp.int32)
    state0_smem = jnp.array([[0, K_total, 0, 0, 0, 0, 0, 0]], dtype=jnp.int32)
    state0_vec = jnp.zeros((8, 128), jnp.int32)

    h1 = _sc_hist_pass(acts_i32, state0_vec, None, 19, 4096)
    st1_smem, st1_vec = _scan(h1, state0_smem, 1 << 19, 4096)
    h2 = _sc_hist_pass(acts_i32, st1_vec, 19, 7, 4096)
    st2_smem, st2_vec = _scan(h2, st1_smem, 1 << 7, 4096)
    h3 = _sc_hist_pass(acts_i32, st2_vec, 7, 0, 128)
    st3_smem, _ = _scan(h3, st2_smem, 1, 128)

    BT2 = min(1024, B)
    KT2 = min(2048, D)
    out = pl.pallas_call(
        _decode_kernel,
        grid=(B // BT2, D // KT2),
        in_specs=[
            pl.BlockSpec(memory_space=pltpu.SMEM),
            pl.BlockSpec((BT2, KT2), lambda i, k: (i, k)),
            pl.BlockSpec((A, KT2), lambda i, k: (0, k)),
            pl.BlockSpec((1, A), lambda i, k: (0, 0)),
        ],
        out_specs=pl.BlockSpec((BT2, A), lambda i, k: (i, 0)),
        out_shape=jax.ShapeDtypeStruct((B, A), jnp.float32),
    )(st3_smem, acts, W_dec, b_dec.reshape(1, A))
    return out


# speedup vs baseline: 4.0774x; 4.0774x over previous
import functools

import jax
import jax.numpy as jnp
from jax import lax
from jax.experimental import pallas as pl
from jax.experimental.pallas import tpu as pltpu
from jax.experimental.pallas import tpu_sc as plsc

K_PER_ROW = 64
NC = 2
NS = 16
NW = NC * NS


def _encode_kernel(x_ref, w_ref, be_ref, bd_ref, out_ref):
    xb = x_ref[...] - bd_ref[...]
    acc = lax.dot_general(
        xb, w_ref[...], (((1,), (1,)), ((), ())),
        preferred_element_type=jnp.float32,
    )
    out_ref[...] = jnp.maximum(acc + be_ref[...], 0.0)


def _sc_hist_body(match_shift, bin_shift, nbins, rows_per_w, row_words,
                  acts, state, hists,
                  buf0, buf1, lo_v, hist_v, merged_v, sem0, sem1):
    c = lax.axis_index("c")
    s = lax.axis_index("s")
    wid = s * NC + c
    base_row = wid * rows_per_w

    pltpu.sync_copy(state.at[0, pl.ds(0, 16)], lo_v)
    lo_vec = lo_v[...]

    zeros16 = jnp.zeros((16,), jnp.int32)
    ones16 = jnp.ones((16,), jnp.int32)
    iota16 = lax.iota(jnp.int32, 16)

    @functools.partial(plsc.parallel_loop, 0, nbins, unroll=8)
    def _zero(i):
        hist_v[pl.ds(i * 16, 16)] = zeros16

    def start(buf, sem, step):
        st = jnp.minimum(step, rows_per_w - 1)
        pltpu.make_async_copy(acts.at[base_row + st], buf, sem).start()

    def wait(buf, sem, step):
        st = jnp.minimum(step, rows_per_w - 1)
        pltpu.make_async_copy(acts.at[base_row + st], buf, sem).wait()

    bin_mask = (nbins - 1) << 4

    def process(buf):
        @functools.partial(plsc.parallel_loop, 0, row_words // 16,
                           unroll=16)
        def _proc(i):
            v = buf[pl.ds(i * 16, 16)]
            if bin_shift >= 4:
                idx = ((v >> (bin_shift - 4)) & bin_mask) + iota16
            else:
                idx = ((v << (4 - bin_shift)) & bin_mask) + iota16
            if match_shift is None:
                plsc.addupdate_scatter(hist_v, [idx], ones16)
            else:
                m = ((v ^ lo_vec) >> match_shift) == 0
                plsc.addupdate_scatter(hist_v, [idx], ones16, mask=m)

    start(buf0, sem0, 0)
    start(buf1, sem1, 1)

    def obody(g, _):
        step0 = g * 2
        wait(buf0, sem0, step0)
        process(buf0)
        start(buf0, sem0, step0 + 2)
        wait(buf1, sem1, step0 + 1)
        process(buf1)
        start(buf1, sem1, step0 + 3)
        return 0

    lax.fori_loop(0, rows_per_w // 2, obody, 0)
    wait(buf0, sem0, rows_per_w - 1)
    wait(buf1, sem1, rows_per_w - 1)

    @functools.partial(plsc.parallel_loop, 0, nbins // 16, unroll=4)
    def _merge(g):
        bidx = g * 256 + iota16 * 16
        acc = plsc.load_gather(hist_v, [bidx])
        for l in range(1, 16):
            acc = acc + plsc.load_gather(hist_v, [bidx + l])
        merged_v[pl.ds(g * 16, 16)] = acc

    pltpu.sync_copy(merged_v, hists.at[wid])


def _sc_hist_pass(acts_i32, state_vec, match_shift, bin_shift, nbins):
    B, D = acts_i32.shape
    rows_per_w = B // NW
    mesh = plsc.VectorSubcoreMesh(core_axis_name="c", subcore_axis_name="s")
    fn = functools.partial(
        pl.kernel,
        out_type=jax.ShapeDtypeStruct((NW, nbins), jnp.int32),
        mesh=mesh,
        compiler_params=pltpu.CompilerParams(needs_layout_passes=False),
        scratch_types=[
            pltpu.VMEM((D,), jnp.int32),
            pltpu.VMEM((D,), jnp.int32),
            pltpu.VMEM((16,), jnp.int32),
            pltpu.VMEM((nbins * 16,), jnp.int32),
            pltpu.VMEM((nbins,), jnp.int32),
            pltpu.SemaphoreType.DMA,
            pltpu.SemaphoreType.DMA,
        ],
    )(functools.partial(_sc_hist_body, match_shift, bin_shift, nbins,
                        rows_per_w, D))
    return fn(acts_i32, state_vec)


def _scan_kernel(width, nbins, hists_ref, state_ref, new_smem_ref, new_vec_ref):
    lo = state_ref[0, 0]
    K = state_ref[0, 1]
    h = jnp.sum(hists_ref[...], axis=0)
    R = nbins // 128
    h2f = h.reshape(R, 128).astype(jnp.float32)
    cmaskf = (
        lax.broadcasted_iota(jnp.int32, (128, 128), 0)
        >= lax.broadcasted_iota(jnp.int32, (128, 128), 1)
    ).astype(jnp.float32)
    ws = lax.dot_general(
        h2f, cmaskf, (((1,), (0,)), ((), ())),
        precision=lax.Precision.HIGHEST,
        preferred_element_type=jnp.float32,
    )
    rowtot = jnp.sum(h2f, axis=1, keepdims=True)
    rmaskf = (
        lax.broadcasted_iota(jnp.int32, (R, R), 1)
        > lax.broadcasted_iota(jnp.int32, (R, R), 0)
    ).astype(jnp.float32)
    rs = lax.dot_general(
        rmaskf, rowtot, (((1,), (0,)), ((), ())),
        precision=lax.Precision.HIGHEST,
        preferred_element_type=jnp.float32,
    )
    S = ws + rs
    Kf = K.astype(jnp.float32)
    b = jnp.sum((S >= Kf).astype(jnp.int32)) - 1
    s_next = jnp.maximum(
        jnp.max(jnp.where(S < Kf, S, -1.0)).astype(jnp.int32), 0
    )
    new_lo = lo + b * width
    new_k = K - s_next
    new_smem_ref[0, 0] = new_lo
    new_smem_ref[0, 1] = new_k
    new_vec_ref[...] = jnp.full((8, 128), new_lo, jnp.int32)


def _scan(hists, state_smem, width, nbins):
    return pl.pallas_call(
        functools.partial(_scan_kernel, width, nbins),
        in_specs=[
            pl.BlockSpec((NW, nbins), lambda: (0, 0)),
            pl.BlockSpec(memory_space=pltpu.SMEM),
        ],
        out_specs=[
            pl.BlockSpec(memory_space=pltpu.SMEM),
            pl.BlockSpec((8, 128), lambda: (0, 0)),
        ],
        out_shape=[
            jax.ShapeDtypeStruct((1, 8), jnp.int32),
            jax.ShapeDtypeStruct((8, 128), jnp.int32),
        ],
    )(hists, state_smem)


def _decode_kernel(thr_ref, acts_ref, w_ref, bd_ref, out_ref):
    k = pl.program_id(1)
    thr = thr_ref[0, 0]
    a = acts_ref[...]
    bits = lax.bitcast_convert_type(a, jnp.int32)
    enc = jnp.where(bits >= thr, a, 0.0)
    part = lax.dot_general(
        enc, w_ref[...], (((1,), (1,)), ((), ())),
        preferred_element_type=jnp.float32,
    )

    @pl.when(k == 0)
    def _first():
        out_ref[...] = part + bd_ref[...]

    @pl.when(k != 0)
    def _acc():
        out_ref[...] += part


def kernel(x, W_enc, b_enc, W_dec, b_dec):
    B, A = x.shape
    D = W_enc.shape[0]
    K_total = K_PER_ROW * B

    BT = min(512, B)
    DT = min(2048, D)
    acts = pl.pallas_call(
        _encode_kernel,
        grid=(D // DT, B // BT),
        in_specs=[
            pl.BlockSpec((BT, A), lambda j, i: (i, 0)),
            pl.BlockSpec((DT, A), lambda j, i: (j, 0)),
            pl.BlockSpec((1, DT), lambda j, i: (0, j)),
            pl.BlockSpec((1, A), lambda j, i: (0, 0)),
        ],
        out_specs=pl.BlockSpec((BT, DT), lambda j, i: (i, j)),
        out_shape=jax.ShapeDtypeStruct((B, D), jnp.float32),
    )(x, W_enc, b_enc.reshape(1, D), b_dec.reshape(1, A))

    acts_i32 = lax.bitcast_convert_type(acts, jnp.int32)
    state0_smem = jnp.array([[0, K_total, 0, 0, 0, 0, 0, 0]], dtype=jnp.int32)
    state0_vec = jnp.zeros((8, 128), jnp.int32)

    h1 = _sc_hist_pass(acts_i32, state0_vec, None, 19, 4096)
    st1_smem, st1_vec = _scan(h1, state0_smem, 1 << 19, 4096)
    h2 = _sc_hist_pass(acts_i32, st1_vec, 19, 7, 4096)
    st2_smem, st2_vec = _scan(h2, st1_smem, 1 << 7, 4096)
    h3 = _sc_hist_pass(acts_i32, st2_vec, 7, 0, 128)
    st3_smem, _ = _scan(h3, st2_smem, 1, 128)

    BT2 = min(1024, B)
    KT2 = min(2048, D)
    out = pl.pallas_call(
        _decode_kernel,
        grid=(B // BT2, D // KT2),
        in_specs=[
            pl.BlockSpec(memory_space=pltpu.SMEM),
            pl.BlockSpec((BT2, KT2), lambda i, k: (i, k)),
            pl.BlockSpec((A, KT2), lambda i, k: (0, k)),
            pl.BlockSpec((1, A), lambda i, k: (0, 0)),
        ],
        out_specs=pl.BlockSpec((BT2, A), lambda i, k: (i, 0)),
        out_shape=jax.ShapeDtypeStruct((B, A), jnp.float32),
    )(st3_smem, acts, W_dec, b_dec.reshape(1, A))
    return out
